# baseline (speedup 1.0000x reference)
import jax
import jax.numpy as jnp
from jax.experimental import pallas as pl
from jax.experimental.pallas import tpu as pltpu

_B, _D, _A = 2048, 128, 512
_MAX_T = 17
_HORIZON = 2
_CG = 256
_NCG = _B // _CG
_C = 256
_R = _B // 128


def _rollout_body(state_ref, act_ref, wt_ref, bt_ref, aemb_ref, wp_ref,
                  out_ref, st_ref, emb0_ref):
    f32 = jnp.float32
    i32 = jnp.int32
    W_t = wt_ref[...]
    b_t1 = bt_ref[...]
    A_emb = aemb_ref[...]
    W_p = wp_ref[...]

    eye128 = (jax.lax.broadcasted_iota(i32, (128, 128), 0) ==
              jax.lax.broadcasted_iota(i32, (128, 128), 1)).astype(f32)
    cum_u = (jax.lax.broadcasted_iota(i32, (128, 128), 0) <=
             jax.lax.broadcasted_iota(i32, (128, 128), 1)).astype(f32)
    excl_l = (jax.lax.broadcasted_iota(i32, (_R, _R), 1) <
              jax.lax.broadcasted_iota(i32, (_R, _R), 0)).astype(f32)
    ones128 = jnp.ones((128, 1), f32)
    ones1c = jnp.ones((1, _C), f32)
    k_iota = jax.lax.broadcasted_iota(i32, (_C, 128), 0).astype(f32)
    sub_ac = jax.lax.broadcasted_iota(i32, (_A, _C), 0)
    sub_ag = jax.lax.broadcasted_iota(i32, (_A, _CG), 0)

    def tr(x):
        return jax.lax.dot_general(x, eye128, (((0,), (0,)), ((), ())),
                                   preferred_element_type=f32)

    def argfirst0(x, iota, width):
        mx = jnp.max(x, axis=0, keepdims=True)
        return jnp.min(jnp.where(x == mx, iota, width), axis=0,
                       keepdims=True)

    def pw_code(sT):
        p = (sT[1:2, :] > sT[0:1, :]).astype(f32)
        a, b, c = sT[2:3, :], sT[3:4, :], sT[4:5, :]
        w1 = (b > a) & (b >= c)
        w2 = (c > a) & (c > b)
        w = jnp.where(w1, f32(1.0), f32(0.0)) + jnp.where(w2, f32(2.0),
                                                          f32(0.0))
        return 4.0 * p + w

    def in_progress_f(code_now, code_init):
        return (jnp.abs(code_now - code_init) == 4.0).astype(i32)

    def transition(sT, embT):
        return jax.nn.sigmoid(
            jax.lax.dot_general(W_t, sT, (((0,), (0,)), ((), ())),
                                preferred_element_type=f32) + embT + b_t1)

    W_cat = jnp.concatenate([W_p, W_t], axis=1)

    def logits_and_pre(sT):
        both = jax.lax.dot_general(W_cat, sT, (((0,), (0,)), ((), ())),
                                   preferred_element_type=f32)
        return both[:_A, :], both[_A:, :]

    act = act_ref[...]

    for _step in range(_HORIZON):
        code_gs, m_gs = [], []
        for g in range(_NCG):
            cols = pl.ds(g * _CG, _CG)
            if _step == 0:
                sT = jnp.concatenate(
                    [tr(state_ref[pl.ds(g * _CG + j * 128, 128), :])
                     for j in range(_CG // 128)], axis=1)
                oh0 = (sub_ag ==
                       act[:, g * _CG:(g + 1) * _CG]).astype(f32)
                e0 = jax.lax.dot_general(
                    A_emb, oh0, (((0,), (0,)), ((), ())),
                    preferred_element_type=f32)
                emb0_ref[:, cols] = e0
            else:
                sT = st_ref[:, cols]
                e0 = emb0_ref[:, cols]
            code_g = pw_code(sT)
            sT = transition(sT, e0)
            m_gs.append(in_progress_f(pw_code(sT), code_g).astype(f32))
            st_ref[:, cols] = sT
            code_gs.append(code_g)
        code_full = jnp.concatenate(code_gs, axis=1)
        m_full = jnp.concatenate(m_gs, axis=1)

        def defl_cond(c):
            t0, m = c
            return (t0 < _MAX_T) & (jnp.sum(m) > f32(_C))

        def defl_body(c):
            t0, m = c
            new_ms = []
            for g in range(_NCG):
                cols = pl.ds(g * _CG, _CG)
                sT = st_ref[:, cols]
                p_logits, pre = logits_and_pre(sT)
                aidx = argfirst0(p_logits, sub_ag, _A)
                oh = (sub_ag == aidx).astype(f32)
                emb = jax.lax.dot_general(
                    A_emb, oh, (((0,), (0,)), ((), ())),
                    preferred_element_type=f32)
                ns = jax.nn.sigmoid(pre + emb + b_t1)
                s2 = jnp.where(m[:, g * _CG:(g + 1) * _CG] > 0, ns, sT)
                st_ref[:, cols] = s2
                new_ms.append(in_progress_f(
                    pw_code(s2),
                    code_full[:, g * _CG:(g + 1) * _CG]).astype(f32))
            return t0 + 1, jnp.concatenate(new_ms, axis=1)

        t0, m_full = jax.lax.while_loop(defl_cond, defl_body,
                                        (jnp.int32(0), m_full))

        m_lane = jnp.concatenate(
            [m_full[:, r * 128:(r + 1) * 128] for r in range(_R)], axis=0)
        c_within = jnp.dot(m_lane, cum_u, preferred_element_type=f32)
        tot = jnp.dot(m_lane, ones128, preferred_element_type=f32)
        offs = jax.lax.dot_general(excl_l, tot, (((1,), (0,)), ((), ())),
                                   preferred_element_type=f32)
        pos_lane = c_within + offs - 1.0
        n_act = jnp.sum(tot).astype(i32)

        def make_onehots(start):
            s_f = start.astype(f32)
            ohs = []
            for r in range(_R):
                rel = pos_lane[r:r + 1, :] - s_f
                hit = (k_iota == rel) & (m_lane[r:r + 1, :] > 0.0)
                ohs.append(hit.astype(f32))
            return ohs

        def chunk_body(start):
            ohs = make_onehots(start)
            comp = None
            code_c = None
            for r in range(_R):
                cols = pl.ds(r * 128, 128)
                pc = jax.lax.dot_general(
                    st_ref[:, cols], ohs[r], (((1,), (1,)), ((), ())),
                    preferred_element_type=f32)
                cc = jax.lax.dot_general(
                    code_full[:, r * 128:(r + 1) * 128], ohs[r],
                    (((1,), (1,)), ((), ())), preferred_element_type=f32)
                comp = pc if comp is None else comp + pc
                code_c = cc if code_c is None else code_c + cc

            def cond(c):
                t, _s, m = c
                return (t < _MAX_T) & (jnp.max(m) > 0)

            def body(c):
                t, sT, m = c
                p_logits, pre = logits_and_pre(sT)
                aidx = argfirst0(p_logits, sub_ac, _A)
                oh = (sub_ac == aidx).astype(f32)
                emb = jax.lax.dot_general(
                    A_emb, oh, (((0,), (0,)), ((), ())),
                    preferred_element_type=f32)
                ns = jax.nn.sigmoid(pre + emb + b_t1)
                s2 = jnp.where(m > 0, ns, sT)
                return t + 1, s2, in_progress_f(pw_code(s2), code_c)

            m_init = in_progress_f(pw_code(comp), code_c)
            _, comp, _ = jax.lax.while_loop(
                cond, body, (t0, comp, m_init))

            for r in range(_R):
                cols = pl.ds(r * 128, 128)
                scat = jax.lax.dot_general(
                    comp, ohs[r], (((1,), (0,)), ((), ())),
                    preferred_element_type=f32)
                memb = jax.lax.dot_general(
                    ones1c, ohs[r], (((1,), (0,)), ((), ())),
                    preferred_element_type=f32)
                st_ref[:, cols] = jnp.where(memb > 0.0, scat,
                                            st_ref[:, cols])
            return start + _C

        jax.lax.while_loop(lambda s: (s < n_act) & (t0 < _MAX_T),
                           chunk_body, jnp.int32(0))

    for r in range(_R):
        out_ref[pl.ds(r * 128, 128), :] = tr(st_ref[:, pl.ds(r * 128, 128)])


@jax.jit
def kernel(state, action, W_t, b_t, A_emb, W_p):
    act2 = action.reshape(1, _B)
    bt1 = b_t.reshape(_D, 1)
    return pl.pallas_call(
        _rollout_body,
        grid=(1,),
        in_specs=[
            pl.BlockSpec((_B, _D), lambda i: (0, 0)),
            pl.BlockSpec((1, _B), lambda i: (0, 0)),
            pl.BlockSpec((_D, _D), lambda i: (0, 0)),
            pl.BlockSpec((_D, 1), lambda i: (0, 0)),
            pl.BlockSpec((_A, _D), lambda i: (0, 0)),
            pl.BlockSpec((_D, _A), lambda i: (0, 0)),
        ],
        out_specs=pl.BlockSpec((_B, _D), lambda i: (0, 0)),
        out_shape=jax.ShapeDtypeStruct((_B, _D), jnp.float32),
        scratch_shapes=[
            pltpu.VMEM((_D, _B), jnp.float32),
            pltpu.VMEM((_D, _B), jnp.float32),
        ],
    )(state, act2, W_t, bt1, A_emb, W_p)


# speedup vs baseline: 1.2576x; 1.2576x over previous
import jax
import jax.numpy as jnp
from jax.experimental import pallas as pl
from jax.experimental.pallas import tpu as pltpu

_B, _D, _A = 2048, 128, 512
_MAX_T = 17
_HORIZON = 2
_CG = 256
_NCG = _B // _CG
_C = 256
_R = _B // 128


def _rollout_body(state_ref, act_ref, wt_ref, bt_ref, aemb_ref, wp_ref,
                  out_ref, st_ref, emb0_ref):
    f32 = jnp.float32
    i32 = jnp.int32
    W_t = wt_ref[...]
    b_t1 = bt_ref[...]
    A_emb = aemb_ref[...]
    W_p = wp_ref[...]

    eye128 = (jax.lax.broadcasted_iota(i32, (128, 128), 0) ==
              jax.lax.broadcasted_iota(i32, (128, 128), 1)).astype(f32)
    cum_u = (jax.lax.broadcasted_iota(i32, (128, 128), 0) <=
             jax.lax.broadcasted_iota(i32, (128, 128), 1)).astype(f32)
    excl_l = (jax.lax.broadcasted_iota(i32, (_R, _R), 1) <
              jax.lax.broadcasted_iota(i32, (_R, _R), 0)).astype(f32)
    ones128 = jnp.ones((128, 1), f32)
    ones1c = jnp.ones((1, _C), f32)
    k_iota = jax.lax.broadcasted_iota(i32, (_C, 128), 0).astype(f32)
    sub_ac = jax.lax.broadcasted_iota(i32, (_A, _C), 0)
    sub_ag = jax.lax.broadcasted_iota(i32, (_A, _CG), 0)

    def tr(x):
        return jax.lax.dot_general(x, eye128, (((0,), (0,)), ((), ())),
                                   preferred_element_type=f32)

    def argfirst0(x, iota, width):
        mx = jnp.max(x, axis=0, keepdims=True)
        return jnp.min(jnp.where(x == mx, iota, width), axis=0,
                       keepdims=True)

    def pw_code(sT):
        p = (sT[1:2, :] > sT[0:1, :]).astype(f32)
        a, b, c = sT[2:3, :], sT[3:4, :], sT[4:5, :]
        w1 = (b > a) & (b >= c)
        w2 = (c > a) & (c > b)
        w = jnp.where(w1, f32(1.0), f32(0.0)) + jnp.where(w2, f32(2.0),
                                                          f32(0.0))
        return 4.0 * p + w

    def in_progress_f(code_now, code_init):
        return (jnp.abs(code_now - code_init) == 4.0).astype(i32)

    def transition(sT, embT):
        return jax.nn.sigmoid(
            jax.lax.dot_general(W_t, sT, (((0,), (0,)), ((), ())),
                                preferred_element_type=f32) + embT + b_t1)

    W_cat = jnp.concatenate([W_p, W_t], axis=1)

    def logits_and_pre(sT):
        both = jax.lax.dot_general(W_cat, sT, (((0,), (0,)), ((), ())),
                                   preferred_element_type=f32)
        return both[:_A, :], both[_A:, :]

    for r in range(_R):
        st_ref[:, pl.ds(r * 128, 128)] = tr(state_ref[pl.ds(r * 128, 128), :])
    for g in range(_NCG):
        cols = pl.ds(g * _CG, _CG)
        oh0 = (sub_ag == act_ref[:, cols]).astype(f32)
        emb0_ref[:, cols] = jax.lax.dot_general(
            A_emb, oh0, (((0,), (0,)), ((), ())), preferred_element_type=f32)

    for _step in range(_HORIZON):
        code_gs, m_gs = [], []
        for g in range(_NCG):
            cols = pl.ds(g * _CG, _CG)
            sT = st_ref[:, cols]
            code_g = pw_code(sT)
            sT = transition(sT, emb0_ref[:, cols])
            m_gs.append(in_progress_f(pw_code(sT), code_g).astype(f32))
            st_ref[:, cols] = sT
            code_gs.append(code_g)
        code_full = jnp.concatenate(code_gs, axis=1)
        m_full = jnp.concatenate(m_gs, axis=1)

        def defl_cond(c):
            t0, m = c
            return (t0 < _MAX_T) & (jnp.sum(m) > f32(_C))

        def defl_body(c):
            t0, m = c
            new_ms = []
            for g in range(_NCG):
                cols = pl.ds(g * _CG, _CG)
                sT = st_ref[:, cols]
                p_logits, pre = logits_and_pre(sT)
                aidx = argfirst0(p_logits, sub_ag, _A)
                oh = (sub_ag == aidx).astype(f32)
                emb = jax.lax.dot_general(
                    A_emb, oh, (((0,), (0,)), ((), ())),
                    preferred_element_type=f32)
                ns = jax.nn.sigmoid(pre + emb + b_t1)
                s2 = jnp.where(m[:, g * _CG:(g + 1) * _CG] > 0, ns, sT)
                st_ref[:, cols] = s2
                new_ms.append(in_progress_f(
                    pw_code(s2),
                    code_full[:, g * _CG:(g + 1) * _CG]).astype(f32))
            return t0 + 1, jnp.concatenate(new_ms, axis=1)

        t0, m_full = jax.lax.while_loop(defl_cond, defl_body,
                                        (jnp.int32(0), m_full))

        m_lane = jnp.concatenate(
            [m_full[:, r * 128:(r + 1) * 128] for r in range(_R)], axis=0)
        c_within = jnp.dot(m_lane, cum_u, preferred_element_type=f32)
        tot = jnp.dot(m_lane, ones128, preferred_element_type=f32)
        offs = jax.lax.dot_general(excl_l, tot, (((1,), (0,)), ((), ())),
                                   preferred_element_type=f32)
        pos_lane = c_within + offs - 1.0
        n_act = jnp.sum(tot).astype(i32)

        def make_onehots(start):
            s_f = start.astype(f32)
            ohs = []
            for r in range(_R):
                rel = pos_lane[r:r + 1, :] - s_f
                hit = (k_iota == rel) & (m_lane[r:r + 1, :] > 0.0)
                ohs.append(hit.astype(f32))
            return ohs

        def chunk_body(start):
            ohs = make_onehots(start)
            comp = None
            code_c = None
            for r in range(_R):
                cols = pl.ds(r * 128, 128)
                pc = jax.lax.dot_general(
                    st_ref[:, cols], ohs[r], (((1,), (1,)), ((), ())),
                    preferred_element_type=f32)
                cc = jax.lax.dot_general(
                    code_full[:, r * 128:(r + 1) * 128], ohs[r],
                    (((1,), (1,)), ((), ())), preferred_element_type=f32)
                comp = pc if comp is None else comp + pc
                code_c = cc if code_c is None else code_c + cc

            def body(_t, c):
                sT, m = c
                p_logits, pre = logits_and_pre(sT)
                aidx = argfirst0(p_logits, sub_ac, _A)
                oh = (sub_ac == aidx).astype(f32)
                emb = jax.lax.dot_general(
                    A_emb, oh, (((0,), (0,)), ((), ())),
                    preferred_element_type=f32)
                ns = jax.nn.sigmoid(pre + emb + b_t1)
                s2 = jnp.where(m > 0, ns, sT)
                return s2, in_progress_f(pw_code(s2), code_c)

            m_init = in_progress_f(pw_code(comp), code_c)
            comp, _ = jax.lax.fori_loop(t0, _MAX_T, body, (comp, m_init))

            for r in range(_R):
                cols = pl.ds(r * 128, 128)
                scat = jax.lax.dot_general(
                    comp, ohs[r], (((1,), (0,)), ((), ())),
                    preferred_element_type=f32)
                memb = jax.lax.dot_general(
                    ones1c, ohs[r], (((1,), (0,)), ((), ())),
                    preferred_element_type=f32)
                st_ref[:, cols] = jnp.where(memb > 0.0, scat,
                                            st_ref[:, cols])
            return start + _C

        jax.lax.while_loop(lambda s: (s < n_act) & (t0 < _MAX_T),
                           chunk_body, jnp.int32(0))

    for r in range(_R):
        out_ref[pl.ds(r * 128, 128), :] = tr(st_ref[:, pl.ds(r * 128, 128)])


@jax.jit
def kernel(state, action, W_t, b_t, A_emb, W_p):
    act2 = action.reshape(1, _B)
    bt1 = b_t.reshape(_D, 1)
    return pl.pallas_call(
        _rollout_body,
        grid=(1,),
        in_specs=[
            pl.BlockSpec((_B, _D), lambda i: (0, 0)),
            pl.BlockSpec((1, _B), lambda i: (0, 0)),
            pl.BlockSpec((_D, _D), lambda i: (0, 0)),
            pl.BlockSpec((_D, 1), lambda i: (0, 0)),
            pl.BlockSpec((_A, _D), lambda i: (0, 0)),
            pl.BlockSpec((_D, _A), lambda i: (0, 0)),
        ],
        out_specs=pl.BlockSpec((_B, _D), lambda i: (0, 0)),
        out_shape=jax.ShapeDtypeStruct((_B, _D), jnp.float32),
        scratch_shapes=[
            pltpu.VMEM((_D, _B), jnp.float32),
            pltpu.VMEM((_D, _B), jnp.float32),
        ],
    )(state, act2, W_t, bt1, A_emb, W_p)
